# Initial kernel scaffold; baseline (speedup 1.0000x reference)
import functools

import jax
import jax.numpy as jnp
from jax import lax
from jax.experimental import pallas as pl
from jax.experimental.pallas import tpu as pltpu
from jax.experimental.pallas import tpu_sc as plsc

NU, NV, D, DS = 25000, 25000, 128, 64
S, NNZ, E = 5, 80000, 400000
HG, HS, HE, NB, NC = 160, 64, 64, 3, 5
H32 = HG // S

NW = 32
NTILE = 16
ACC_ROWS = 25088
RPT = ACC_ROWS // NTILE
ZROWS = 196
CH = 128
NPHASE = 2 * S


def _enc_matmul(x, w_enc):
    n = x.shape[0]
    r = 1000

    def body(x_ref, w_ref, o_ref):
        xb = x_ref[...]
        for i in range(S):
            o_ref[i] = jnp.dot(xb, w_ref[i], preferred_element_type=jnp.float32)

    return pl.pallas_call(
        body,
        grid=(n // r,),
        in_specs=[
            pl.BlockSpec((r, D), lambda g: (g, 0)),
            pl.BlockSpec((S, D, H32), lambda g: (0, 0, 0)),
        ],
        out_specs=pl.BlockSpec((S, r, H32), lambda g: (0, g, 0)),
        out_shape=jax.ShapeDtypeStruct((S, n, H32), jnp.float32),
    )(x, w_enc)


def _seg_sum_sc(tmp_u, tmp_v, sup_row, sup_col, sup_val):
    mesh = plsc.VectorSubcoreMesh(core_axis_name="c", subcore_axis_name="s")
    nchunk = NNZ // CH

    @functools.partial(
        pl.kernel,
        mesh=mesh,
        out_type=jax.ShapeDtypeStruct((2, NPHASE * ACC_ROWS, H32), jnp.float32),
        scratch_types=[
            pltpu.VMEM((ZROWS, H32), jnp.float32),
            pltpu.VMEM((CH,), jnp.int32),
            pltpu.VMEM((CH,), jnp.int32),
            pltpu.VMEM((CH,), jnp.float32),
            pltpu.VMEM((CH, H32), jnp.float32),
            pltpu.VMEM_SHARED((ACC_ROWS, H32), jnp.float32),
            pltpu.SemaphoreType.DMA,
        ],
    )
    def k(tu_hbm, tv_hbm, row_hbm, col_hbm, val_hbm, out_hbm,
          zbuf, gidx, sidx, vals, rows, acc, sem):
        cid = lax.axis_index("c")
        sid = lax.axis_index("s")
        wid = sid * 2 + cid
        nch = (nchunk - wid + NW - 1) // NW

        def zb(zr, carry):
            zbuf[zr, pl.ds(0, 16)] = jnp.zeros((16,), jnp.float32)
            zbuf[zr, pl.ds(16, 16)] = jnp.zeros((16,), jnp.float32)
            return carry
        lax.fori_loop(0, ZROWS, zb, 0)

        base_r = sid * RPT
        for p in range(NPHASE):
            i = p % S
            user_side = p < S
            tab = tv_hbm if user_side else tu_hbm
            gsrc = col_hbm if user_side else row_hbm
            ssrc = row_hbm if user_side else col_hbm

            for zi in range(RPT // ZROWS):
                pltpu.sync_copy(zbuf, acc.at[pl.ds(base_r + zi * ZROWS, ZROWS)])
            plsc.subcore_barrier()

            def chunk(j, carry):
                e0 = (j * NW + wid) * CH
                pltpu.sync_copy(gsrc.at[i, pl.ds(e0, CH)], gidx)
                for v8 in range(CH // 16):
                    gidx[pl.ds(v8 * 16, 16)] = (
                        gidx[pl.ds(v8 * 16, 16)] + (i * NU))
                pltpu.async_copy(tab.at[gidx], rows, sem).wait()
                pltpu.sync_copy(val_hbm.at[i, pl.ds(e0, CH)], vals)
                pltpu.sync_copy(ssrc.at[i, pl.ds(e0, CH)], sidx)

                def mul(e, c2):
                    sv = vals[e]
                    rows[e, pl.ds(0, 16)] = rows[e, pl.ds(0, 16)] * sv
                    rows[e, pl.ds(16, 16)] = rows[e, pl.ds(16, 16)] * sv
                    return c2
                lax.fori_loop(0, CH, mul, 0)
                pltpu.sync_copy(rows, acc.at[sidx], add=True)
                return carry
            lax.fori_loop(0, nch, chunk, 0)
            plsc.subcore_barrier()
            pltpu.sync_copy(
                acc.at[pl.ds(base_r, RPT)],
                out_hbm.at[cid, pl.ds(p * ACC_ROWS + base_r, RPT)])
            plsc.subcore_barrier()

    return k(tmp_u, tmp_v, sup_row, sup_col, sup_val)


def _embed(h, side, w1, b1, w2, side_sel):
    n = side.shape[0]
    r = 1000

    def body(h_ref, side_ref, w1_ref, b1_ref, w2_ref, o_ref):
        sh = jax.nn.relu(
            jnp.dot(side_ref[...], w1_ref[...],
                    preferred_element_type=jnp.float32) + b1_ref[...])
        acc = jnp.dot(sh, w2_ref[HG:, :], preferred_element_type=jnp.float32)
        for i in range(S):
            g = jax.nn.relu(h_ref[0, i] + h_ref[1, i])
            acc = acc + jnp.dot(g, w2_ref[i * H32:(i + 1) * H32, :],
                                preferred_element_type=jnp.float32)
        o_ref[...] = acc

    return pl.pallas_call(
        body,
        grid=(n // r,),
        in_specs=[
            pl.BlockSpec((2, S, r, H32), lambda g: (0, side_sel, g, 0)),
            pl.BlockSpec((r, DS), lambda g: (g, 0)),
            pl.BlockSpec((DS, HS), lambda g: (0, 0)),
            pl.BlockSpec((1, HS), lambda g: (0, 0)),
            pl.BlockSpec((HG + HS, HE), lambda g: (0, 0)),
        ],
        out_specs=pl.BlockSpec((r, HE), lambda g: (g, 0)),
        out_shape=jax.ShapeDtypeStruct((n, HE), jnp.float32),
    )(h, side, w1, b1, w2)


def _edge_gather_sc(user_embed, item_embed, uidx, vidx):
    mesh = plsc.VectorSubcoreMesh(core_axis_name="c", subcore_axis_name="s")
    nchunk = E // CH

    @functools.partial(
        pl.kernel,
        mesh=mesh,
        out_type=(jax.ShapeDtypeStruct((E, HE), jnp.float32),
                  jax.ShapeDtypeStruct((E, HE), jnp.float32)),
        scratch_types=[
            pltpu.VMEM((CH,), jnp.int32),
            pltpu.VMEM((CH, HE), jnp.float32),
            pltpu.SemaphoreType.DMA,
        ],
    )
    def k(ue_hbm, ie_hbm, uidx_hbm, vidx_hbm, u_out, v_out, idxb, rowsb, sem):
        cid = lax.axis_index("c")
        sid = lax.axis_index("s")
        wid = sid * 2 + cid
        nch = (nchunk - wid + NW - 1) // NW
        for side in range(2):
            tab = ue_hbm if side == 0 else ie_hbm
            isrc = uidx_hbm if side == 0 else vidx_hbm
            dst = u_out if side == 0 else v_out

            def chunk(j, carry):
                e0 = (j * NW + wid) * CH
                pltpu.sync_copy(isrc.at[pl.ds(e0, CH)], idxb)
                pltpu.async_copy(tab.at[idxb], rowsb, sem).wait()
                pltpu.sync_copy(rowsb, dst.at[pl.ds(e0, CH)])
                return carry
            lax.fori_loop(0, nch, chunk, 0)

    return k(user_embed, item_embed, uidx, vidx)


def _decode(u_rows, v_rows, w_dec, w_cls):
    b = 2000

    def body(u_ref, v_ref, wd_ref, wc_ref, o_ref):
        u = u_ref[...]
        v = v_ref[...]
        cols = []
        for i in range(NB):
            t = jnp.dot(u, wd_ref[i], preferred_element_type=jnp.float32)
            cols.append(jnp.sum(t * v, axis=1, keepdims=True))
        basis = jnp.concatenate(cols, axis=1)
        o_ref[...] = jnp.dot(basis, wc_ref[...],
                             preferred_element_type=jnp.float32)

    return pl.pallas_call(
        body,
        grid=(E // b,),
        in_specs=[
            pl.BlockSpec((b, HE), lambda g: (g, 0)),
            pl.BlockSpec((b, HE), lambda g: (g, 0)),
            pl.BlockSpec((NB, HE, HE), lambda g: (0, 0, 0)),
            pl.BlockSpec((NB, NC), lambda g: (0, 0)),
        ],
        out_specs=pl.BlockSpec((b, NC), lambda g: (g, 0)),
        out_shape=jax.ShapeDtypeStruct((E, NC), jnp.float32),
    )(u_rows, v_rows, w_dec, w_cls)


def kernel(user_inputs, item_inputs, user_side_inputs, item_side_inputs,
           sup_row, sup_col, sup_val, user_edge_idx, item_edge_idx,
           W_enc, W1u, b1u, W1v, b1v, W2u, W2v, W_dec, W_cls):
    sup_row = sup_row.astype(jnp.int32)
    sup_col = sup_col.astype(jnp.int32)
    user_edge_idx = user_edge_idx.astype(jnp.int32)
    item_edge_idx = item_edge_idx.astype(jnp.int32)

    tmp_u = _enc_matmul(user_inputs, W_enc).reshape(S * NU, H32)
    tmp_v = _enc_matmul(item_inputs, W_enc).reshape(S * NV, H32)

    h = _seg_sum_sc(tmp_u, tmp_v, sup_row, sup_col, sup_val)
    h = h.reshape(2, NPHASE, ACC_ROWS, H32)

    user_embed = _embed(h, user_side_inputs, W1u, b1u.reshape(1, HS), W2u, 0)
    item_embed = _embed(h, item_side_inputs, W1v, b1v.reshape(1, HS), W2v, 1)

    u_rows, v_rows = _edge_gather_sc(user_embed, item_embed,
                                     user_edge_idx, item_edge_idx)
    return _decode(u_rows, v_rows, W_dec, W_cls)


# speedup vs baseline: 2.5349x; 2.5349x over previous
import functools

import jax
import jax.numpy as jnp
from jax import lax
from jax.experimental import pallas as pl
from jax.experimental.pallas import tpu as pltpu
from jax.experimental.pallas import tpu_sc as plsc

NU, NV, D, DS = 25000, 25000, 128, 64
S, NNZ, E = 5, 80000, 400000
HG, HS, HE, NB, NC = 160, 64, 64, 3, 5
H32 = HG // S

NW = 32
NTILE = 16
ACC_ROWS = 25088
RPT = ACC_ROWS // NTILE
ZROWS = 196
CH = 128
NPHASE = 2 * S


def _enc_matmul(x, w_enc):
    n = x.shape[0]
    r = 1000

    def body(x_ref, w_ref, o_ref):
        xb = x_ref[...]
        for i in range(S):
            o_ref[i] = jnp.dot(xb, w_ref[i], preferred_element_type=jnp.float32)

    return pl.pallas_call(
        body,
        grid=(n // r,),
        in_specs=[
            pl.BlockSpec((r, D), lambda g: (g, 0)),
            pl.BlockSpec((S, D, H32), lambda g: (0, 0, 0)),
        ],
        out_specs=pl.BlockSpec((S, r, H32), lambda g: (0, g, 0)),
        out_shape=jax.ShapeDtypeStruct((S, n, H32), jnp.float32),
    )(x, w_enc)


def _seg_sum_sc(tmp_u, tmp_v, sup_row, sup_col, sup_val):
    mesh = plsc.VectorSubcoreMesh(core_axis_name="c", subcore_axis_name="s")
    nchunk = NNZ // CH

    @functools.partial(
        pl.kernel,
        mesh=mesh,
        out_type=jax.ShapeDtypeStruct((2, NPHASE * ACC_ROWS, H32), jnp.float32),
        compiler_params=pltpu.CompilerParams(use_tc_tiling_on_sc=False),
        scratch_types=[
            pltpu.VMEM((ZROWS, H32), jnp.float32),
            pltpu.VMEM((CH,), jnp.int32),
            pltpu.VMEM((CH,), jnp.int32),
            pltpu.VMEM((CH,), jnp.float32),
            pltpu.VMEM((CH, H32), jnp.float32),
            pltpu.VMEM_SHARED((ACC_ROWS, H32), jnp.float32),
            pltpu.SemaphoreType.DMA,
        ],
    )
    def k(tu_hbm, tv_hbm, row_hbm, col_hbm, val_hbm, out_hbm,
          zbuf, gidx, sidx, vals, rows, acc, sem):
        cid = lax.axis_index("c")
        sid = lax.axis_index("s")
        wid = sid * 2 + cid
        nch = (nchunk - wid + NW - 1) // NW

        def zb(zr, carry):
            zbuf[zr, pl.ds(0, 16)] = jnp.zeros((16,), jnp.float32)
            zbuf[zr, pl.ds(16, 16)] = jnp.zeros((16,), jnp.float32)
            return carry
        lax.fori_loop(0, ZROWS, zb, 0)

        base_r = sid * RPT
        for p in range(NPHASE):
            i = p % S
            user_side = p < S
            tab = tv_hbm if user_side else tu_hbm
            gsrc = col_hbm if user_side else row_hbm
            ssrc = row_hbm if user_side else col_hbm

            for zi in range(RPT // ZROWS):
                pltpu.sync_copy(zbuf, acc.at[pl.ds(base_r + zi * ZROWS, ZROWS)])
            plsc.subcore_barrier()

            def chunk(j, carry):
                e0 = i * NNZ + (j * NW + wid) * CH
                pltpu.sync_copy(gsrc.at[pl.ds(e0, CH)], gidx)
                for v8 in range(CH // 16):
                    gidx[pl.ds(v8 * 16, 16)] = (
                        gidx[pl.ds(v8 * 16, 16)] + (i * NU))
                pltpu.async_copy(tab.at[gidx], rows, sem).wait()
                pltpu.sync_copy(val_hbm.at[pl.ds(e0, CH)], vals)
                pltpu.sync_copy(ssrc.at[pl.ds(e0, CH)], sidx)

                def mul(g, c2):
                    vv = vals[pl.ds(g * 16, 16)]
                    for t in range(16):
                        e = g * 16 + t
                        sv = vv[t]
                        rows[e, pl.ds(0, 16)] = rows[e, pl.ds(0, 16)] * sv
                        rows[e, pl.ds(16, 16)] = rows[e, pl.ds(16, 16)] * sv
                    return c2
                lax.fori_loop(0, CH // 16, mul, 0)
                pltpu.sync_copy(rows, acc.at[sidx], add=True)
                return carry
            lax.fori_loop(0, nch, chunk, 0)
            plsc.subcore_barrier()
            pltpu.sync_copy(
                acc.at[pl.ds(base_r, RPT)],
                out_hbm.at[cid, pl.ds(p * ACC_ROWS + base_r, RPT)])
            plsc.subcore_barrier()

    return k(tmp_u, tmp_v, sup_row, sup_col, sup_val)


def _embed(h, side, w1, b1, w2, side_sel):
    n = side.shape[0]
    r = 1000

    def body(h_ref, side_ref, w1_ref, b1_ref, w2_ref, o_ref):
        sh = jax.nn.relu(
            jnp.dot(side_ref[...], w1_ref[...],
                    preferred_element_type=jnp.float32) + b1_ref[...])
        acc = jnp.dot(sh, w2_ref[HG:, :], preferred_element_type=jnp.float32)
        for i in range(S):
            g = jax.nn.relu(h_ref[0, i] + h_ref[1, i])
            acc = acc + jnp.dot(g, w2_ref[i * H32:(i + 1) * H32, :],
                                preferred_element_type=jnp.float32)
        o_ref[...] = acc

    return pl.pallas_call(
        body,
        grid=(n // r,),
        in_specs=[
            pl.BlockSpec((2, S, r, H32), lambda g: (0, side_sel, g, 0)),
            pl.BlockSpec((r, DS), lambda g: (g, 0)),
            pl.BlockSpec((DS, HS), lambda g: (0, 0)),
            pl.BlockSpec((1, HS), lambda g: (0, 0)),
            pl.BlockSpec((HG + HS, HE), lambda g: (0, 0)),
        ],
        out_specs=pl.BlockSpec((r, HE), lambda g: (g, 0)),
        out_shape=jax.ShapeDtypeStruct((n, HE), jnp.float32),
    )(h, side, w1, b1, w2)


def _edge_gather_sc(user_embed, item_embed, uidx, vidx):
    mesh = plsc.VectorSubcoreMesh(core_axis_name="c", subcore_axis_name="s")
    nchunk = E // CH

    @functools.partial(
        pl.kernel,
        mesh=mesh,
        out_type=(jax.ShapeDtypeStruct((E, HE), jnp.float32),
                  jax.ShapeDtypeStruct((E, HE), jnp.float32)),
        compiler_params=pltpu.CompilerParams(use_tc_tiling_on_sc=False),
        scratch_types=[
            pltpu.VMEM((CH,), jnp.int32),
            pltpu.VMEM((CH, HE), jnp.float32),
            pltpu.SemaphoreType.DMA,
        ],
    )
    def k(ue_hbm, ie_hbm, uidx_hbm, vidx_hbm, u_out, v_out, idxb, rowsb, sem):
        cid = lax.axis_index("c")
        sid = lax.axis_index("s")
        wid = sid * 2 + cid
        nch = (nchunk - wid + NW - 1) // NW
        for side in range(2):
            tab = ue_hbm if side == 0 else ie_hbm
            isrc = uidx_hbm if side == 0 else vidx_hbm
            dst = u_out if side == 0 else v_out

            def chunk(j, carry):
                e0 = (j * NW + wid) * CH
                pltpu.sync_copy(isrc.at[pl.ds(e0, CH)], idxb)
                pltpu.async_copy(tab.at[idxb], rowsb, sem).wait()
                pltpu.sync_copy(rowsb, dst.at[pl.ds(e0, CH)])
                return carry
            lax.fori_loop(0, nch, chunk, 0)

    return k(user_embed, item_embed, uidx, vidx)


def _decode(u_rows, v_rows, w_dec, w_cls):
    b = 2000

    def body(u_ref, v_ref, wd_ref, wc_ref, o_ref):
        u = u_ref[...]
        v = v_ref[...]
        cols = []
        for i in range(NB):
            t = jnp.dot(u, wd_ref[i], preferred_element_type=jnp.float32)
            cols.append(jnp.sum(t * v, axis=1, keepdims=True))
        basis = jnp.concatenate(cols, axis=1)
        o_ref[...] = jnp.dot(basis, wc_ref[...],
                             preferred_element_type=jnp.float32)

    return pl.pallas_call(
        body,
        grid=(E // b,),
        in_specs=[
            pl.BlockSpec((b, HE), lambda g: (g, 0)),
            pl.BlockSpec((b, HE), lambda g: (g, 0)),
            pl.BlockSpec((NB, HE, HE), lambda g: (0, 0, 0)),
            pl.BlockSpec((NB, NC), lambda g: (0, 0)),
        ],
        out_specs=pl.BlockSpec((b, NC), lambda g: (g, 0)),
        out_shape=jax.ShapeDtypeStruct((E, NC), jnp.float32),
    )(u_rows, v_rows, w_dec, w_cls)


def kernel(user_inputs, item_inputs, user_side_inputs, item_side_inputs,
           sup_row, sup_col, sup_val, user_edge_idx, item_edge_idx,
           W_enc, W1u, b1u, W1v, b1v, W2u, W2v, W_dec, W_cls):
    sup_row = sup_row.astype(jnp.int32)
    sup_col = sup_col.astype(jnp.int32)
    user_edge_idx = user_edge_idx.astype(jnp.int32)
    item_edge_idx = item_edge_idx.astype(jnp.int32)

    tmp_u = _enc_matmul(user_inputs, W_enc).reshape(S * NU, H32)
    tmp_v = _enc_matmul(item_inputs, W_enc).reshape(S * NV, H32)

    h = _seg_sum_sc(tmp_u, tmp_v, sup_row.reshape(-1), sup_col.reshape(-1),
                    sup_val.reshape(-1))
    h = h.reshape(2, NPHASE, ACC_ROWS, H32)

    user_embed = _embed(h, user_side_inputs, W1u, b1u.reshape(1, HS), W2u, 0)
    item_embed = _embed(h, item_side_inputs, W1v, b1v.reshape(1, HS), W2v, 1)

    u_rows, v_rows = _edge_gather_sc(user_embed, item_embed,
                                     user_edge_idx, item_edge_idx)
    return _decode(u_rows, v_rows, W_dec, W_cls)
